# Initial kernel scaffold; baseline (speedup 1.0000x reference)
import functools

import jax
import jax.numpy as jnp
from jax import lax
from jax.experimental import pallas as pl
from jax.experimental.pallas import tpu as pltpu
from jax.experimental.pallas import tpu_sc as plsc

NC = 2
NS = 16
L = 16
NW = NC * NS

E = 320000
N = 10000
D = 128
C = 80
PER_W = E // NW
NCHUNK = PER_W // C
DV = D // L


def _dot16(bs, bc, j):
    acc = bs[j, pl.ds(0, L)] * bc[j, pl.ds(0, L)]
    for q in range(1, DV):
        acc = acc + bs[j, pl.ds(q * L, L)] * bc[j, pl.ds(q * L, L)]
    return jnp.sum(acc)


def _make_decode():
    mesh = plsc.VectorSubcoreMesh(core_axis_name="c", subcore_axis_name="s")

    @functools.partial(
        pl.kernel,
        mesh=mesh,
        out_type=jax.ShapeDtypeStruct((E,), jnp.float32),
        scratch_types=[
            pltpu.VMEM((PER_W,), jnp.int32),
            pltpu.VMEM((PER_W,), jnp.int32),
            pltpu.VMEM((C, D), jnp.float32),
            pltpu.VMEM((C, D), jnp.float32),
            pltpu.VMEM((C, D), jnp.float32),
            pltpu.VMEM((C, D), jnp.float32),
            pltpu.VMEM((PER_W,), jnp.float32),
            pltpu.SemaphoreType.DMA,
            pltpu.SemaphoreType.DMA,
            pltpu.SemaphoreType.DMA,
            pltpu.SemaphoreType.DMA,
        ],
    )
    def decode(zs_hbm, zc_hbm, si_hbm, di_hbm, out_hbm,
               idx_s, idx_d, bs0, bc0, bs1, bc1, out_v,
               sem_s0, sem_c0, sem_s1, sem_c1):
        wid = lax.axis_index("s") * NC + lax.axis_index("c")
        base = wid * PER_W
        pltpu.sync_copy(si_hbm.at[pl.ds(base, PER_W)], idx_s)
        pltpu.sync_copy(di_hbm.at[pl.ds(base, PER_W)], idx_d)

        lane = lax.iota(jnp.int32, 16)

        def start(i, bs, bc, sem_s, sem_c):
            pltpu.async_copy(zs_hbm.at[idx_s.at[pl.ds(i * C, C)]], bs, sem_s)
            pltpu.async_copy(zc_hbm.at[idx_d.at[pl.ds(i * C, C)]], bc, sem_c)

        def wait(bs, bc, sem_s, sem_c):
            pltpu.make_async_copy(zs_hbm.at[pl.ds(0, C)], bs, sem_s).wait()
            pltpu.make_async_copy(zc_hbm.at[pl.ds(0, C)], bc, sem_c).wait()

        def compute(i, bs, bc):
            for g in range(C // 16):
                vec = jnp.zeros((16,), jnp.float32)
                for jj in range(16):
                    s = _dot16(bs, bc, g * 16 + jj)
                    vec = jnp.where(lane == jj, jnp.full((16,), s), vec)
                sig = 1.0 / (1.0 + jnp.exp(-vec))
                out_v[pl.ds(i * C + g * 16, 16)] = sig

        start(0, bs0, bc0, sem_s0, sem_c0)

        def body(g, carry):
            i = 2 * g
            start(i + 1, bs1, bc1, sem_s1, sem_c1)
            wait(bs0, bc0, sem_s0, sem_c0)
            compute(i, bs0, bc0)
            start(i + 2, bs0, bc0, sem_s0, sem_c0)
            wait(bs1, bc1, sem_s1, sem_c1)
            compute(i + 1, bs1, bc1)
            return carry

        lax.fori_loop(0, (NCHUNK - 1) // 2, body, 0)
        wait(bs0, bc0, sem_s0, sem_c0)
        compute(NCHUNK - 1, bs0, bc0)

        pltpu.sync_copy(out_v, out_hbm.at[pl.ds(base, PER_W)])

    return decode


_decode = _make_decode()


def kernel(z_source, z_child, edge_index):
    src_idx = edge_index[0].astype(jnp.int32)
    dst_idx = edge_index[1].astype(jnp.int32)
    return _decode(z_source, z_child, src_idx, dst_idx)


# speedup vs baseline: 3.7434x; 3.7434x over previous
import functools

import jax
import jax.numpy as jnp
from jax import lax
from jax.experimental import pallas as pl
from jax.experimental.pallas import tpu as pltpu
from jax.experimental.pallas import tpu_sc as plsc

NC = 2
NS = 16
L = 16
NW = NC * NS

E = 320000
N = 10000
D = 128
C = 80
PER_W = E // NW
NCHUNK = PER_W // C
DV = D // L


_GATHER_DNUMS = lax.GatherDimensionNumbers(
    offset_dims=(), collapsed_slice_dims=(0,), start_index_map=(0,))


def _lane_perm(v, idx16):
    return lax.gather(v, idx16[:, None], _GATHER_DNUMS, (1,),
                      mode=lax.GatherScatterMode.PROMISE_IN_BOUNDS)


def _dot16(bs, bc, j, perms):
    acc = bs[j, pl.ds(0, L)] * bc[j, pl.ds(0, L)]
    for q in range(1, DV):
        acc = acc + bs[j, pl.ds(q * L, L)] * bc[j, pl.ds(q * L, L)]
    for p in perms:
        acc = acc + _lane_perm(acc, p)
    return acc


def _make_decode():
    mesh = plsc.VectorSubcoreMesh(core_axis_name="c", subcore_axis_name="s")

    @functools.partial(
        pl.kernel,
        mesh=mesh,
        out_type=jax.ShapeDtypeStruct((E,), jnp.float32),
        scratch_types=[
            pltpu.VMEM((PER_W,), jnp.int32),
            pltpu.VMEM((PER_W,), jnp.int32),
            pltpu.VMEM((C, D), jnp.float32),
            pltpu.VMEM((C, D), jnp.float32),
            pltpu.VMEM((C, D), jnp.float32),
            pltpu.VMEM((C, D), jnp.float32),
            pltpu.VMEM((PER_W,), jnp.float32),
            pltpu.SemaphoreType.DMA,
            pltpu.SemaphoreType.DMA,
            pltpu.SemaphoreType.DMA,
            pltpu.SemaphoreType.DMA,
        ],
    )
    def decode(zs_hbm, zc_hbm, si_hbm, di_hbm, out_hbm,
               idx_s, idx_d, bs0, bc0, bs1, bc1, out_v,
               sem_s0, sem_c0, sem_s1, sem_c1):
        wid = lax.axis_index("s") * NC + lax.axis_index("c")
        base = wid * PER_W
        pltpu.sync_copy(si_hbm.at[pl.ds(base, PER_W)], idx_s)
        pltpu.sync_copy(di_hbm.at[pl.ds(base, PER_W)], idx_d)

        lane = lax.iota(jnp.int32, 16)
        perms = [lane ^ k for k in (1, 2, 4, 8)]

        def start(i, bs, bc, sem_s, sem_c):
            pltpu.async_copy(zs_hbm.at[idx_s.at[pl.ds(i * C, C)]], bs, sem_s)
            pltpu.async_copy(zc_hbm.at[idx_d.at[pl.ds(i * C, C)]], bc, sem_c)

        def wait(bs, bc, sem_s, sem_c):
            pltpu.make_async_copy(zs_hbm.at[pl.ds(0, C)], bs, sem_s).wait()
            pltpu.make_async_copy(zc_hbm.at[pl.ds(0, C)], bc, sem_c).wait()

        def compute(i, bs, bc):
            for g in range(C // 16):
                vec = jnp.zeros((16,), jnp.float32)
                for jj in range(16):
                    s = _dot16(bs, bc, g * 16 + jj, perms)
                    vec = jnp.where(lane == jj, s, vec)
                sig = 1.0 / (1.0 + jnp.exp(-vec))
                out_v[pl.ds(i * C + g * 16, 16)] = sig

        start(0, bs0, bc0, sem_s0, sem_c0)

        def body(g, carry):
            i = 2 * g
            start(i + 1, bs1, bc1, sem_s1, sem_c1)
            wait(bs0, bc0, sem_s0, sem_c0)
            compute(i, bs0, bc0)
            start(i + 2, bs0, bc0, sem_s0, sem_c0)
            wait(bs1, bc1, sem_s1, sem_c1)
            compute(i + 1, bs1, bc1)
            return carry

        lax.fori_loop(0, (NCHUNK - 1) // 2, body, 0)
        wait(bs0, bc0, sem_s0, sem_c0)
        compute(NCHUNK - 1, bs0, bc0)

        pltpu.sync_copy(out_v, out_hbm.at[pl.ds(base, PER_W)])

    return decode


_decode = _make_decode()


def kernel(z_source, z_child, edge_index):
    src_idx = edge_index[0].astype(jnp.int32)
    dst_idx = edge_index[1].astype(jnp.int32)
    return _decode(z_source, z_child, src_idx, dst_idx)


# speedup vs baseline: 5.6329x; 1.5048x over previous
import functools

import jax
import jax.numpy as jnp
from jax import lax
from jax.experimental import pallas as pl
from jax.experimental.pallas import tpu as pltpu
from jax.experimental.pallas import tpu_sc as plsc

NC = 2
NS = 16
L = 16
NW = NC * NS

E = 320000
N = 10000
D = 128
C = 80
PER_W = E // NW
NCHUNK = PER_W // C

_GATHER_DNUMS = lax.GatherDimensionNumbers(
    offset_dims=(), collapsed_slice_dims=(0,), start_index_map=(0,))


def _lane_perm(v, idx16):
    return lax.gather(v, idx16[:, None], _GATHER_DNUMS, (1,),
                      mode=lax.GatherScatterMode.PROMISE_IN_BOUNDS)


def _treesum(vs):
    while len(vs) > 1:
        vs = [a + b for a, b in zip(vs[0::2], vs[1::2])]
    return vs[0]


def _unpack2(w):
    hi = plsc.bitcast(w & jnp.int32(-65536), jnp.float32)
    lo = plsc.bitcast(w << 16, jnp.float32)
    return hi, lo


def _edge_partials(bs, bc, j):
    prods = []
    for q in range(D // 32):
        hs, ls = _unpack2(bs[j, pl.ds(q * L, L)])
        hc, lc = _unpack2(bc[j, pl.ds(q * L, L)])
        prods.append(hs * hc)
        prods.append(ls * lc)
    return _treesum(prods)


def _merge16(accs, lane, perms):
    for stage, k in enumerate((1, 2, 4, 8)):
        pk = perms[stage]
        even = (lane & k) == 0
        nxt = []
        for x, y in zip(accs[0::2], accs[1::2]):
            sx = x + _lane_perm(x, pk)
            sy = y + _lane_perm(y, pk)
            nxt.append(jnp.where(even, sx, sy))
        accs = nxt
    return accs[0]


def _make_decode():
    mesh = plsc.VectorSubcoreMesh(core_axis_name="c", subcore_axis_name="s")

    @functools.partial(
        pl.kernel,
        mesh=mesh,
        compiler_params=pltpu.CompilerParams(
            needs_layout_passes=False, use_tc_tiling_on_sc=False),
        out_type=jax.ShapeDtypeStruct((E,), jnp.float32),
        scratch_types=[
            pltpu.VMEM((PER_W,), jnp.int32),
            pltpu.VMEM((PER_W,), jnp.int32),
            pltpu.VMEM((C, D // 2), jnp.int32),
            pltpu.VMEM((C, D // 2), jnp.int32),
            pltpu.VMEM((C, D // 2), jnp.int32),
            pltpu.VMEM((C, D // 2), jnp.int32),
            pltpu.VMEM((PER_W,), jnp.float32),
            pltpu.SemaphoreType.DMA,
            pltpu.SemaphoreType.DMA,
            pltpu.SemaphoreType.DMA,
            pltpu.SemaphoreType.DMA,
        ],
    )
    def decode(zs_hbm, zc_hbm, si_hbm, di_hbm, out_hbm,
               idx_s, idx_d, bs0, bc0, bs1, bc1, out_v,
               sem_s0, sem_c0, sem_s1, sem_c1):
        wid = lax.axis_index("s") * NC + lax.axis_index("c")
        base = wid * PER_W
        pltpu.sync_copy(si_hbm.at[pl.ds(base, PER_W)], idx_s)
        pltpu.sync_copy(di_hbm.at[pl.ds(base, PER_W)], idx_d)

        lane = lax.iota(jnp.int32, 16)
        perms = [lane ^ k for k in (1, 2, 4, 8)]

        def start(i, bs, bc, sem_s, sem_c):
            pltpu.async_copy(zs_hbm.at[idx_s.at[pl.ds(i * C, C)]], bs, sem_s)
            pltpu.async_copy(zc_hbm.at[idx_d.at[pl.ds(i * C, C)]], bc, sem_c)

        def wait(bs, bc, sem_s, sem_c):
            pltpu.make_async_copy(zs_hbm.at[pl.ds(0, C)], bs, sem_s).wait()
            pltpu.make_async_copy(zc_hbm.at[pl.ds(0, C)], bc, sem_c).wait()

        def compute(i, bs, bc):
            for g in range(C // 16):
                accs = [_edge_partials(bs, bc, g * 16 + jj) for jj in range(16)]
                vec = _merge16(accs, lane, perms)
                sig = 1.0 / (1.0 + jnp.exp(-vec))
                out_v[pl.ds(i * C + g * 16, 16)] = sig

        start(0, bs0, bc0, sem_s0, sem_c0)

        def body(g, carry):
            i = 2 * g
            start(i + 1, bs1, bc1, sem_s1, sem_c1)
            wait(bs0, bc0, sem_s0, sem_c0)
            compute(i, bs0, bc0)
            start(i + 2, bs0, bc0, sem_s0, sem_c0)
            wait(bs1, bc1, sem_s1, sem_c1)
            compute(i + 1, bs1, bc1)
            return carry

        lax.fori_loop(0, (NCHUNK - 1) // 2, body, 0)
        wait(bs0, bc0, sem_s0, sem_c0)
        compute(NCHUNK - 1, bs0, bc0)

        pltpu.sync_copy(out_v, out_hbm.at[pl.ds(base, PER_W)])

    return decode


_decode = _make_decode()


def kernel(z_source, z_child, edge_index):
    src_idx = edge_index[0].astype(jnp.int32)
    dst_idx = edge_index[1].astype(jnp.int32)
    zs = lax.bitcast_convert_type(
        z_source.astype(jnp.bfloat16).reshape(N, D // 2, 2), jnp.int32)
    zc = lax.bitcast_convert_type(
        z_child.astype(jnp.bfloat16).reshape(N, D // 2, 2), jnp.int32)
    return _decode(zs, zc, src_idx, dst_idx)


# speedup vs baseline: 7.4499x; 1.3226x over previous
import functools

import jax
import jax.numpy as jnp
from jax import lax
from jax.experimental import pallas as pl
from jax.experimental.pallas import tpu as pltpu
from jax.experimental.pallas import tpu_sc as plsc

NC = 2
NS = 16
L = 16
NW = NC * NS

E = 320000
N = 10000
D = 128
C = 80
PER_W = E // NW
NCHUNK = PER_W // C

_GATHER_DNUMS = lax.GatherDimensionNumbers(
    offset_dims=(), collapsed_slice_dims=(0,), start_index_map=(0,))


def _lane_perm(v, idx16):
    return lax.gather(v, idx16[:, None], _GATHER_DNUMS, (1,),
                      mode=lax.GatherScatterMode.PROMISE_IN_BOUNDS)


def _treesum(vs):
    while len(vs) > 1:
        vs = [a + b for a, b in zip(vs[0::2], vs[1::2])]
    return vs[0]


def _edge_partials(bs, bc, j):
    ps = [bs[j, pl.ds(q * 32, 32)] * bc[j, pl.ds(q * 32, 32)]
          for q in range(D // 32)]
    s = _treesum(ps)
    a, b = plsc.unpack(s, format=plsc.PackFormat.INTERLEAVED)
    return a + b


def _merge16(accs, lane, perms):
    for stage, k in enumerate((1, 2, 4, 8)):
        pk = perms[stage]
        even = (lane & k) == 0
        nxt = []
        for x, y in zip(accs[0::2], accs[1::2]):
            sx = x + _lane_perm(x, pk)
            sy = y + _lane_perm(y, pk)
            nxt.append(jnp.where(even, sx, sy))
        accs = nxt
    return accs[0]


def _make_decode():
    mesh = plsc.VectorSubcoreMesh(core_axis_name="c", subcore_axis_name="s")

    @functools.partial(
        pl.kernel,
        mesh=mesh,
        compiler_params=pltpu.CompilerParams(
            needs_layout_passes=False, use_tc_tiling_on_sc=False),
        out_type=jax.ShapeDtypeStruct((E,), jnp.float32),
        scratch_types=[
            pltpu.VMEM((PER_W,), jnp.int32),
            pltpu.VMEM((PER_W,), jnp.int32),
            pltpu.VMEM((C, D), jnp.bfloat16),
            pltpu.VMEM((C, D), jnp.bfloat16),
            pltpu.VMEM((C, D), jnp.bfloat16),
            pltpu.VMEM((C, D), jnp.bfloat16),
            pltpu.VMEM((PER_W,), jnp.float32),
            pltpu.SemaphoreType.DMA,
            pltpu.SemaphoreType.DMA,
            pltpu.SemaphoreType.DMA,
            pltpu.SemaphoreType.DMA,
        ],
    )
    def decode(zs_hbm, zc_hbm, si_hbm, di_hbm, out_hbm,
               idx_s, idx_d, bs0, bc0, bs1, bc1, out_v,
               sem_s0, sem_c0, sem_s1, sem_c1):
        wid = lax.axis_index("s") * NC + lax.axis_index("c")
        base = wid * PER_W
        pltpu.sync_copy(si_hbm.at[pl.ds(base, PER_W)], idx_s)
        pltpu.sync_copy(di_hbm.at[pl.ds(base, PER_W)], idx_d)

        lane = lax.iota(jnp.int32, 16)
        perms = [lane ^ k for k in (1, 2, 4, 8)]

        def start(i, bs, bc, sem_s, sem_c):
            pltpu.async_copy(zs_hbm.at[idx_s.at[pl.ds(i * C, C)]], bs, sem_s)
            pltpu.async_copy(zc_hbm.at[idx_d.at[pl.ds(i * C, C)]], bc, sem_c)

        def wait(bs, bc, sem_s, sem_c):
            pltpu.make_async_copy(zs_hbm.at[pl.ds(0, C)], bs, sem_s).wait()
            pltpu.make_async_copy(zc_hbm.at[pl.ds(0, C)], bc, sem_c).wait()

        def compute(i, bs, bc):
            for g in range(C // 16):
                accs = [_edge_partials(bs, bc, g * 16 + jj) for jj in range(16)]
                vec = _merge16(accs, lane, perms)
                sig = 1.0 / (1.0 + jnp.exp(-vec))
                out_v[pl.ds(i * C + g * 16, 16)] = sig

        start(0, bs0, bc0, sem_s0, sem_c0)

        def body(g, carry):
            i = 2 * g
            start(i + 1, bs1, bc1, sem_s1, sem_c1)
            wait(bs0, bc0, sem_s0, sem_c0)
            compute(i, bs0, bc0)
            start(i + 2, bs0, bc0, sem_s0, sem_c0)
            wait(bs1, bc1, sem_s1, sem_c1)
            compute(i + 1, bs1, bc1)
            return carry

        lax.fori_loop(0, (NCHUNK - 1) // 2, body, 0)
        wait(bs0, bc0, sem_s0, sem_c0)
        compute(NCHUNK - 1, bs0, bc0)

        pltpu.sync_copy(out_v, out_hbm.at[pl.ds(base, PER_W)])

    return decode


_decode = _make_decode()


def kernel(z_source, z_child, edge_index):
    src_idx = edge_index[0].astype(jnp.int32)
    dst_idx = edge_index[1].astype(jnp.int32)
    zs = z_source.astype(jnp.bfloat16)
    zc = z_child.astype(jnp.bfloat16)
    return _decode(zs, zc, src_idx, dst_idx)


# speedup vs baseline: 7.7898x; 1.0456x over previous
import functools

import jax
import jax.numpy as jnp
from jax import lax
from jax.experimental import pallas as pl
from jax.experimental.pallas import tpu as pltpu
from jax.experimental.pallas import tpu_sc as plsc

NC = 2
NS = 16
L = 16
NW = NC * NS

E = 320000
N = 10000
D = 128
C = 80
PER_W = E // NW
NCHUNK = PER_W // C

_GATHER_DNUMS = lax.GatherDimensionNumbers(
    offset_dims=(), collapsed_slice_dims=(0,), start_index_map=(0,))


def _lane_perm(v, idx16):
    return lax.gather(v, idx16[:, None], _GATHER_DNUMS, (1,),
                      mode=lax.GatherScatterMode.PROMISE_IN_BOUNDS)


def _treesum(vs):
    while len(vs) > 1:
        vs = [a + b for a, b in zip(vs[0::2], vs[1::2])]
    return vs[0]


def _edge_partials(bs, bc, j):
    ps = [bs[j, pl.ds(q * 32, 32)] * bc[j, pl.ds(q * 32, 32)]
          for q in range(D // 32)]
    s = _treesum(ps)
    a, b = plsc.unpack(s, format=plsc.PackFormat.INTERLEAVED)
    return a + b


def _combine(x, y, lvl, evens, perms):
    pk = perms[lvl]
    sx = x + _lane_perm(x, pk)
    sy = y + _lane_perm(y, pk)
    return jnp.where(evens[lvl], sx, sy)


def _group16(bs, bc, g, evens, perms):
    stack = []
    for jj in range(16):
        v = _edge_partials(bs, bc, g * 16 + jj)
        lvl = 0
        while stack and stack[-1][0] == lvl:
            _, u = stack.pop()
            v = _combine(u, v, lvl, evens, perms)
            lvl += 1
        stack.append((lvl, v))
    return stack[0][1]


def _make_decode():
    mesh = plsc.VectorSubcoreMesh(core_axis_name="c", subcore_axis_name="s")

    @functools.partial(
        pl.kernel,
        mesh=mesh,
        compiler_params=pltpu.CompilerParams(
            needs_layout_passes=False, use_tc_tiling_on_sc=False),
        out_type=jax.ShapeDtypeStruct((E,), jnp.float32),
        scratch_types=[
            pltpu.VMEM((PER_W,), jnp.int32),
            pltpu.VMEM((PER_W,), jnp.int32),
            pltpu.VMEM((C, D), jnp.bfloat16),
            pltpu.VMEM((C, D), jnp.bfloat16),
            pltpu.VMEM((C, D), jnp.bfloat16),
            pltpu.VMEM((C, D), jnp.bfloat16),
            pltpu.VMEM((PER_W,), jnp.float32),
            pltpu.SemaphoreType.DMA,
            pltpu.SemaphoreType.DMA,
            pltpu.SemaphoreType.DMA,
            pltpu.SemaphoreType.DMA,
        ],
    )
    def decode(zs_hbm, zc_hbm, si_hbm, di_hbm, out_hbm,
               idx_s, idx_d, bs0, bc0, bs1, bc1, out_v,
               sem_s0, sem_c0, sem_s1, sem_c1):
        wid = lax.axis_index("s") * NC + lax.axis_index("c")
        base = wid * PER_W
        pltpu.sync_copy(si_hbm.at[pl.ds(base, PER_W)], idx_s)
        pltpu.sync_copy(di_hbm.at[pl.ds(base, PER_W)], idx_d)

        lane = lax.iota(jnp.int32, 16)
        perms = [lane ^ k for k in (1, 2, 4, 8)]
        evens = [(lane & k) == 0 for k in (1, 2, 4, 8)]

        def start(i, bs, bc, sem_s, sem_c):
            pltpu.async_copy(zs_hbm.at[idx_s.at[pl.ds(i * C, C)]], bs, sem_s)
            pltpu.async_copy(zc_hbm.at[idx_d.at[pl.ds(i * C, C)]], bc, sem_c)

        def wait(bs, bc, sem_s, sem_c):
            pltpu.make_async_copy(zs_hbm.at[pl.ds(0, C)], bs, sem_s).wait()
            pltpu.make_async_copy(zc_hbm.at[pl.ds(0, C)], bc, sem_c).wait()

        def compute(i, bs, bc):
            for g in range(C // 16):
                vec = _group16(bs, bc, g, evens, perms)
                sig = 1.0 / (1.0 + jnp.exp(-vec))
                out_v[pl.ds(i * C + g * 16, 16)] = sig

        start(0, bs0, bc0, sem_s0, sem_c0)

        def body(g, carry):
            i = 2 * g
            start(i + 1, bs1, bc1, sem_s1, sem_c1)
            wait(bs0, bc0, sem_s0, sem_c0)
            compute(i, bs0, bc0)
            start(i + 2, bs0, bc0, sem_s0, sem_c0)
            wait(bs1, bc1, sem_s1, sem_c1)
            compute(i + 1, bs1, bc1)
            return carry

        lax.fori_loop(0, (NCHUNK - 1) // 2, body, 0)
        wait(bs0, bc0, sem_s0, sem_c0)
        compute(NCHUNK - 1, bs0, bc0)

        pltpu.sync_copy(out_v, out_hbm.at[pl.ds(base, PER_W)])

    return decode


_decode = _make_decode()


def kernel(z_source, z_child, edge_index):
    src_idx = edge_index[0].astype(jnp.int32)
    dst_idx = edge_index[1].astype(jnp.int32)
    zs = z_source.astype(jnp.bfloat16)
    zc = z_child.astype(jnp.bfloat16)
    return _decode(zs, zc, src_idx, dst_idx)


# speedup vs baseline: 12.6526x; 1.6243x over previous
import functools

import jax
import jax.numpy as jnp
from jax import lax
from jax.experimental import pallas as pl
from jax.experimental.pallas import tpu as pltpu
from jax.experimental.pallas import tpu_sc as plsc

NC = 2
NS = 16
L = 16
NW = NC * NS

E = 320000
N = 10000
D = 128
C = 80
PER_W = E // NW
NCHUNK = PER_W // C

_GATHER_DNUMS = lax.GatherDimensionNumbers(
    offset_dims=(), collapsed_slice_dims=(0,), start_index_map=(0,))


def _lane_perm(v, idx16):
    return lax.gather(v, idx16[:, None], _GATHER_DNUMS, (1,),
                      mode=lax.GatherScatterMode.PROMISE_IN_BOUNDS)


def _treesum(vs):
    while len(vs) > 1:
        vs = [a + b for a, b in zip(vs[0::2], vs[1::2])]
    return vs[0]


def _edge_partials(bs, bc, j):
    ps = [bs[j, pl.ds(q * 32, 32)] * bc[j, pl.ds(q * 32, 32)]
          for q in range(D // 32)]
    s = _treesum(ps)
    a, b = plsc.unpack(s, format=plsc.PackFormat.INTERLEAVED)
    return a + b


def _combine(x, y, lvl, evens, perms):
    pk = perms[lvl]
    sx = x + _lane_perm(x, pk)
    sy = y + _lane_perm(y, pk)
    return jnp.where(evens[lvl], sx, sy)


def _group16(bs, bc, g, evens, perms):
    stack = []
    for jj in range(16):
        v = _edge_partials(bs, bc, g * 16 + jj)
        lvl = 0
        while stack and stack[-1][0] == lvl:
            _, u = stack.pop()
            v = _combine(u, v, lvl, evens, perms)
            lvl += 1
        stack.append((lvl, v))
    return stack[0][1]


def _make_decode():
    mesh = plsc.VectorSubcoreMesh(core_axis_name="c", subcore_axis_name="s")

    @functools.partial(
        pl.kernel,
        mesh=mesh,
        compiler_params=pltpu.CompilerParams(
            needs_layout_passes=False, use_tc_tiling_on_sc=False),
        out_type=jax.ShapeDtypeStruct((E,), jnp.float32),
        scratch_types=[
            pltpu.VMEM((PER_W,), jnp.int32),
            pltpu.VMEM((PER_W,), jnp.int32),
            pltpu.VMEM((C, D), jnp.bfloat16),
            pltpu.VMEM((C, D), jnp.bfloat16),
            pltpu.VMEM((C, D), jnp.bfloat16),
            pltpu.VMEM((C, D), jnp.bfloat16),
            pltpu.VMEM((C,), jnp.float32),
            pltpu.VMEM((C,), jnp.float32),
            pltpu.VMEM_SHARED((N, D), jnp.bfloat16),
            pltpu.VMEM_SHARED((N, D), jnp.bfloat16),
            pltpu.SemaphoreType.DMA,
            pltpu.SemaphoreType.DMA,
            pltpu.SemaphoreType.DMA,
            pltpu.SemaphoreType.DMA,
            pltpu.SemaphoreType.DMA,
            pltpu.SemaphoreType.DMA,
        ],
    )
    def decode(zs_hbm, zc_hbm, si_hbm, di_hbm, out_hbm,
               idx_s, idx_d, bs0, bc0, bs1, bc1, o0, o1, zs_sh, zc_sh,
               sem_s0, sem_c0, sem_s1, sem_c1, sem_o0, sem_o1):
        wid = lax.axis_index("s") * NC + lax.axis_index("c")
        base = wid * PER_W
        pltpu.sync_copy(si_hbm.at[pl.ds(base, PER_W)], idx_s)
        pltpu.sync_copy(di_hbm.at[pl.ds(base, PER_W)], idx_d)

        sid = lax.axis_index("s")
        rows = N // NS
        r0 = sid * rows
        pltpu.sync_copy(zs_hbm.at[pl.ds(r0, rows)], zs_sh.at[pl.ds(r0, rows)])
        pltpu.sync_copy(zc_hbm.at[pl.ds(r0, rows)], zc_sh.at[pl.ds(r0, rows)])
        plsc.subcore_barrier()

        lane = lax.iota(jnp.int32, 16)
        perms = [lane ^ k for k in (1, 2, 4, 8)]
        evens = [(lane & k) == 0 for k in (1, 2, 4, 8)]

        def start(i, bs, bc, sem_s, sem_c):
            pltpu.async_copy(zs_sh.at[idx_s.at[pl.ds(i * C, C)]], bs, sem_s)
            pltpu.async_copy(zc_sh.at[idx_d.at[pl.ds(i * C, C)]], bc, sem_c)

        def wait(bs, bc, sem_s, sem_c):
            pltpu.make_async_copy(zs_hbm.at[pl.ds(0, C)], bs, sem_s).wait()
            pltpu.make_async_copy(zc_hbm.at[pl.ds(0, C)], bc, sem_c).wait()

        def compute(i, bs, bc, o, sem_o):
            def gbody(g, carry):
                vec = _group16(bs, bc, g, evens, perms)
                sig = 1.0 / (1.0 + jnp.exp(-vec))
                o[pl.ds(g * 16, 16)] = sig
                return carry
            lax.fori_loop(0, C // 16, gbody, 0)
            pltpu.async_copy(o, out_hbm.at[pl.ds(base + i * C, C)], sem_o)

        def wait_out(o, sem_o):
            pltpu.make_async_copy(o, out_hbm.at[pl.ds(0, C)], sem_o).wait()

        start(0, bs0, bc0, sem_s0, sem_c0)
        start(1, bs1, bc1, sem_s1, sem_c1)
        wait(bs0, bc0, sem_s0, sem_c0)
        compute(0, bs0, bc0, o0, sem_o0)
        start(2, bs0, bc0, sem_s0, sem_c0)
        wait(bs1, bc1, sem_s1, sem_c1)
        compute(1, bs1, bc1, o1, sem_o1)

        def body(g, carry):
            i = 2 * g + 2
            start(i + 1, bs1, bc1, sem_s1, sem_c1)
            wait(bs0, bc0, sem_s0, sem_c0)
            wait_out(o0, sem_o0)
            compute(i, bs0, bc0, o0, sem_o0)
            start(i + 2, bs0, bc0, sem_s0, sem_c0)
            wait(bs1, bc1, sem_s1, sem_c1)
            wait_out(o1, sem_o1)
            compute(i + 1, bs1, bc1, o1, sem_o1)
            return carry

        lax.fori_loop(0, (NCHUNK - 3) // 2, body, 0)
        wait(bs0, bc0, sem_s0, sem_c0)
        wait_out(o0, sem_o0)
        compute(NCHUNK - 1, bs0, bc0, o0, sem_o0)
        wait_out(o0, sem_o0)
        wait_out(o1, sem_o1)

    return decode


_decode = _make_decode()


def kernel(z_source, z_child, edge_index):
    src_idx = edge_index[0].astype(jnp.int32)
    dst_idx = edge_index[1].astype(jnp.int32)
    zs = z_source.astype(jnp.bfloat16)
    zc = z_child.astype(jnp.bfloat16)
    return _decode(zs, zc, src_idx, dst_idx)
